# baseline (speedup 1.0000x reference)
import functools

import jax
import jax.numpy as jnp
from jax import lax
from jax.experimental import pallas as pl
from jax.experimental.pallas import tpu as pltpu
from jax.experimental.pallas import tpu_sc as plsc

_N, _C, _H, _W = 128, 64, 56, 56
_SPLIT = 8
_HP, _WP = _H // 2, _W // 2
_NP = _HP * _WP
_Q = 4 * _NP
_CBLK = 2

_KS0 = 0
_KS1 = 42
_KS2 = _KS0 ^ _KS1 ^ 0x1BD11BDA


def _rotl(x, r):
    return (x << jnp.uint32(r)) | (x >> jnp.uint32(32 - r))


def _threefry_bits(t):
    ks = (jnp.uint32(_KS0), jnp.uint32(_KS1), jnp.uint32(_KS2))
    rot = ((13, 15, 26, 6), (17, 29, 16, 24))
    x1 = t + ks[1]
    x0 = x1
    x1 = _rotl(x1, 13) ^ x1
    for r in (15, 26, 6):
        x0 = x0 + x1
        x1 = _rotl(x1, r)
        x1 = x1 ^ x0
    x0 = x0 + ks[1]
    x1 = x1 + ks[2] + jnp.uint32(1)
    for i in range(1, 5):
        for r in rot[i % 2]:
            x0 = x0 + x1
            x1 = _rotl(x1, r)
            x1 = x1 ^ x0
        x0 = x0 + ks[(i + 1) % 3]
        x1 = x1 + ks[(i + 2) % 3] + jnp.uint32(i + 1)
    return x0 ^ x1


def _keys(t):
    return lax.bitcast_convert_type(
        _threefry_bits(lax.bitcast_convert_type(t, jnp.uint32)) >> jnp.uint32(9),
        jnp.int32,
    )


def _ble(x, y):
    return jnp.int32(1) - lax.shift_right_logical(y - x, jnp.int32(31))


def _eq_mask_f32(x, j):
    return (jnp.int32(1) - jnp.minimum(x ^ jnp.int32(j), jnp.int32(1))).astype(
        jnp.float32
    )


def _shuffle_kernel(v_ref, o_ref):
    c0 = _SPLIT + pl.program_id(0) * _CBLK
    shp = (_CBLK, _HP, _WP, _N)
    ci = lax.broadcasted_iota(jnp.int32, shp, 0)
    ai = lax.broadcasted_iota(jnp.int32, shp, 1)
    bi = lax.broadcasted_iota(jnp.int32, shp, 2)
    ni = lax.broadcasted_iota(jnp.int32, shp, 3)
    base = ni * (_C * _Q) + (c0 + ci) * _Q + 112 * ai + 4 * bi

    u = [_keys(base + j) for j in range(4)]

    b01 = (u[0] <= u[1]).astype(jnp.int32)
    b02 = (u[0] <= u[2]).astype(jnp.int32)
    b03 = (u[0] <= u[3]).astype(jnp.int32)
    b12 = (u[1] <= u[2]).astype(jnp.int32)
    b13 = (u[1] <= u[3]).astype(jnp.int32)
    b23 = (u[2] <= u[3]).astype(jnp.int32)
    rank = (
        3 - b01 - b02 - b03,
        2 + b01 - b12 - b13,
        1 + b02 + b12 - b23,
        b03 + b13 + b23,
    )

    v = [v_ref[k] for k in range(4)]
    for j in range(4):
        o = jnp.where(
            rank[0] == j,
            v[0],
            jnp.where(rank[1] == j, v[1], jnp.where(rank[2] == j, v[2], v[3])),
        )
        dh, dw = j // 2, j % 2
        o_ref[:, pl.Slice(dh, _HP, 2), pl.Slice(dw, _WP, 2), :] = o


_SC_MESH = plsc.VectorSubcoreMesh(core_axis_name="c", subcore_axis_name="s")
_PARTS = 4
_ROWS = _HP // _PARTS


def _sc_body(vals_hbm, out_hbm, row_v, buf_v, dma_sem):
    wid = lax.axis_index("s") * 2 + lax.axis_index("c")
    ch = wid // _PARTS
    hq = wid % _PARTS

    niota = lax.broadcasted_iota(jnp.int32, (_N // 8,), 0) * (_C * _Q)

    for a_rel in range(_ROWS):
        a = hq * _ROWS + a_rel
        pltpu.sync_copy(vals_hbm.at[a], row_v)
        tbase_ab0 = ch * _Q + 112 * a

        def b_body(b, _):
            v = [row_v[pl.ds((4 * b + k) * 16, 16)] for k in range(4)]
            tb = tbase_ab0 + 4 * b

            def ng_body(ng, _):
                t = niota + (ng * 16 * (_C * _Q) + tb)
                u = [_keys(t + j) for j in range(4)]
                b01 = _ble(u[0], u[1])
                b02 = _ble(u[0], u[2])
                b03 = _ble(u[0], u[3])
                b12 = _ble(u[1], u[2])
                b13 = _ble(u[1], u[3])
                b23 = _ble(u[2], u[3])
                rank = (
                    3 - b01 - b02 - b03,
                    2 + b01 - b12 - b13,
                    1 + b02 + b12 - b23,
                    b03 + b13 + b23,
                )
                for j in range(4):
                    o = (
                        _eq_mask_f32(rank[0], j) * v[0]
                        + _eq_mask_f32(rank[1], j) * v[1]
                        + _eq_mask_f32(rank[2], j) * v[2]
                        + _eq_mask_f32(rank[3], j) * v[3]
                    )
                    buf_v[j // 2, 2 * b + j % 2, pl.ds(ng * 16, 16)] = o
                return 0

            lax.fori_loop(0, _N // 16, ng_body, 0)
            return 0

        lax.fori_loop(0, _WP, b_body, 0)
        pltpu.sync_copy(buf_v, out_hbm.at[ch, pl.ds(2 * a, 2)])


_sc_shuffle = functools.partial(
    pl.kernel,
    out_type=jax.ShapeDtypeStruct((_SPLIT, _H, _W, _N), jnp.float32),
    mesh=_SC_MESH,
    scratch_types=[
        pltpu.VMEM((_WP * 4 * 16,), jnp.float32),
        pltpu.VMEM((2, _W, _N), jnp.float32),
        pltpu.SemaphoreType.DMA,
    ],
)(_sc_body)


def kernel(T):
    n, c, h, w = T.shape
    vals = T[0, 0].reshape(_HP, 2, _WP, 2).transpose(1, 3, 0, 2)
    vals = vals.reshape(4, _HP, _WP)
    vals_rows = jnp.broadcast_to(
        vals.transpose(1, 2, 0)[:, :, :, None], (_HP, _WP, 4, 16)
    ).reshape(_HP, _WP * 4 * 16)
    vals4 = jnp.broadcast_to(vals[:, :, :, None], (4, _HP, _WP, _N))

    out_sc = _sc_shuffle(vals_rows)

    out_tc = pl.pallas_call(
        _shuffle_kernel,
        out_shape=jax.ShapeDtypeStruct((_C - _SPLIT, _H, _W, _N), jnp.float32),
        grid=((_C - _SPLIT) // _CBLK,),
        in_specs=[pl.BlockSpec((4, _HP, _WP, _N), lambda i: (0, 0, 0, 0))],
        out_specs=pl.BlockSpec((_CBLK, _H, _W, _N), lambda i: (i, 0, 0, 0)),
    )(vals4)

    out = jnp.concatenate([out_sc, out_tc], axis=0)
    return out.transpose(3, 0, 1, 2)


# speedup vs baseline: 1.1308x; 1.1308x over previous
import functools

import jax
import jax.numpy as jnp
from jax import lax
from jax.experimental import pallas as pl
from jax.experimental.pallas import tpu as pltpu
from jax.experimental.pallas import tpu_sc as plsc

_N, _C, _H, _W = 128, 64, 56, 56
_SPLIT = 16
_HP, _WP = _H // 2, _W // 2
_NP = _HP * _WP
_Q = 4 * _NP
_CBLK = 2

_KS0 = 0
_KS1 = 42
_KS2 = _KS0 ^ _KS1 ^ 0x1BD11BDA


def _rotl(x, r):
    return (x << jnp.uint32(r)) | (x >> jnp.uint32(32 - r))


def _threefry_bits(t):
    ks = (jnp.uint32(_KS0), jnp.uint32(_KS1), jnp.uint32(_KS2))
    rot = ((13, 15, 26, 6), (17, 29, 16, 24))
    x1 = t + ks[1]
    x0 = x1
    x1 = _rotl(x1, 13) ^ x1
    for r in (15, 26, 6):
        x0 = x0 + x1
        x1 = _rotl(x1, r)
        x1 = x1 ^ x0
    x0 = x0 + ks[1]
    x1 = x1 + ks[2] + jnp.uint32(1)
    for i in range(1, 5):
        for r in rot[i % 2]:
            x0 = x0 + x1
            x1 = _rotl(x1, r)
            x1 = x1 ^ x0
        x0 = x0 + ks[(i + 1) % 3]
        x1 = x1 + ks[(i + 2) % 3] + jnp.uint32(i + 1)
    return x0 ^ x1


def _keys(t):
    return lax.bitcast_convert_type(
        _threefry_bits(lax.bitcast_convert_type(t, jnp.uint32)) >> jnp.uint32(9),
        jnp.int32,
    )


def _ble(x, y):
    return jnp.int32(1) - lax.shift_right_logical(y - x, jnp.int32(31))


def _eq_mask_f32(x, j):
    return (jnp.int32(1) - jnp.minimum(x ^ jnp.int32(j), jnp.int32(1))).astype(
        jnp.float32
    )


def _shuffle_kernel(v_ref, o_ref):
    c0 = _SPLIT + pl.program_id(0) * _CBLK
    shp = (_CBLK, _HP, _WP, _N)
    ci = lax.broadcasted_iota(jnp.int32, shp, 0)
    ai = lax.broadcasted_iota(jnp.int32, shp, 1)
    bi = lax.broadcasted_iota(jnp.int32, shp, 2)
    ni = lax.broadcasted_iota(jnp.int32, shp, 3)
    base = ni * (_C * _Q) + (c0 + ci) * _Q + 112 * ai + 4 * bi

    u = [_keys(base + j) for j in range(4)]

    b01 = (u[0] <= u[1]).astype(jnp.int32)
    b02 = (u[0] <= u[2]).astype(jnp.int32)
    b03 = (u[0] <= u[3]).astype(jnp.int32)
    b12 = (u[1] <= u[2]).astype(jnp.int32)
    b13 = (u[1] <= u[3]).astype(jnp.int32)
    b23 = (u[2] <= u[3]).astype(jnp.int32)
    rank = (
        3 - b01 - b02 - b03,
        2 + b01 - b12 - b13,
        1 + b02 + b12 - b23,
        b03 + b13 + b23,
    )

    v = [v_ref[k] for k in range(4)]
    for j in range(4):
        o = jnp.where(
            rank[0] == j,
            v[0],
            jnp.where(rank[1] == j, v[1], jnp.where(rank[2] == j, v[2], v[3])),
        )
        dh, dw = j // 2, j % 2
        o_ref[:, pl.Slice(dh, _HP, 2), pl.Slice(dw, _WP, 2), :] = o


_SC_MESH = plsc.VectorSubcoreMesh(core_axis_name="c", subcore_axis_name="s")
_PARTS = 2
_ROWS = _HP // _PARTS


def _sc_body(vals_hbm, out_hbm, row_v, buf_v, dma_sem):
    wid = lax.axis_index("s") * 2 + lax.axis_index("c")
    ch = wid // _PARTS
    hq = wid % _PARTS

    niota = lax.broadcasted_iota(jnp.int32, (_N // 8,), 0) * (_C * _Q)

    for a_rel in range(_ROWS):
        a = hq * _ROWS + a_rel
        pltpu.sync_copy(vals_hbm.at[a], row_v)
        tbase_ab0 = ch * _Q + 112 * a

        def b_body(b, _):
            v = [row_v[pl.ds((4 * b + k) * 16, 16)] for k in range(4)]
            tb = tbase_ab0 + 4 * b

            def ng_body(ng, _):
                t = niota + (ng * 16 * (_C * _Q) + tb)
                u = [_keys(t + j) for j in range(4)]
                b01 = _ble(u[0], u[1])
                b02 = _ble(u[0], u[2])
                b03 = _ble(u[0], u[3])
                b12 = _ble(u[1], u[2])
                b13 = _ble(u[1], u[3])
                b23 = _ble(u[2], u[3])
                rank = (
                    3 - b01 - b02 - b03,
                    2 + b01 - b12 - b13,
                    1 + b02 + b12 - b23,
                    b03 + b13 + b23,
                )
                for j in range(4):
                    o = (
                        _eq_mask_f32(rank[0], j) * v[0]
                        + _eq_mask_f32(rank[1], j) * v[1]
                        + _eq_mask_f32(rank[2], j) * v[2]
                        + _eq_mask_f32(rank[3], j) * v[3]
                    )
                    buf_v[j // 2, 2 * b + j % 2, pl.ds(ng * 16, 16)] = o
                return 0

            lax.fori_loop(0, _N // 16, ng_body, 0)
            return 0

        lax.fori_loop(0, _WP, b_body, 0)
        pltpu.sync_copy(buf_v, out_hbm.at[ch, pl.ds(2 * a, 2)])


_sc_shuffle = functools.partial(
    pl.kernel,
    out_type=jax.ShapeDtypeStruct((_SPLIT, _H, _W, _N), jnp.float32),
    mesh=_SC_MESH,
    scratch_types=[
        pltpu.VMEM((_WP * 4 * 16,), jnp.float32),
        pltpu.VMEM((2, _W, _N), jnp.float32),
        pltpu.SemaphoreType.DMA,
    ],
)(_sc_body)


def kernel(T):
    n, c, h, w = T.shape
    vals = T[0, 0].reshape(_HP, 2, _WP, 2).transpose(1, 3, 0, 2)
    vals = vals.reshape(4, _HP, _WP)
    vals_rows = jnp.broadcast_to(
        vals.transpose(1, 2, 0)[:, :, :, None], (_HP, _WP, 4, 16)
    ).reshape(_HP, _WP * 4 * 16)
    vals4 = jnp.broadcast_to(vals[:, :, :, None], (4, _HP, _WP, _N))

    out_sc = _sc_shuffle(vals_rows)

    out_tc = pl.pallas_call(
        _shuffle_kernel,
        out_shape=jax.ShapeDtypeStruct((_C - _SPLIT, _H, _W, _N), jnp.float32),
        grid=((_C - _SPLIT) // _CBLK,),
        in_specs=[pl.BlockSpec((4, _HP, _WP, _N), lambda i: (0, 0, 0, 0))],
        out_specs=pl.BlockSpec((_CBLK, _H, _W, _N), lambda i: (i, 0, 0, 0)),
    )(vals4)

    out = jnp.concatenate([out_sc, out_tc], axis=0)
    return out.transpose(3, 0, 1, 2)
